# Initial kernel scaffold; baseline (speedup 1.0000x reference)
import jax
import jax.numpy as jnp
from jax.experimental import pallas as pl


def kernel(pos, normals, appr_points, params, contact_idx):
    raise NotImplementedError("write your pallas kernel here")


# speedup vs baseline: 15.9647x; 15.9647x over previous
import functools

import jax
import jax.numpy as jnp
from jax import lax
from jax.experimental import pallas as pl
from jax.experimental.pallas import tpu as pltpu

N = 512
K = 16
NUM_CONTACT = 32
GRIPPER_DEPTH = 0.072 - 0.007
BIG = 1e30


def _dot(a, b):
    return jnp.dot(a, b, preferred_element_type=jnp.float32)


def _gdot(a, b):
    hi = b.astype(jnp.bfloat16).astype(jnp.float32)
    return _dot(a, hi) + _dot(a, b - hi)


def _normalize(v):
    n = jnp.sqrt(jnp.sum(v * v, axis=1, keepdims=True))
    return v / jnp.maximum(n, 1e-12)


def _cross(a, b):
    a0, a1, a2 = a[:, 0:1], a[:, 1:2], a[:, 2:3]
    b0, b1, b2 = b[:, 0:1], b[:, 1:2], b[:, 2:3]
    return jnp.concatenate(
        [a1 * b2 - a2 * b1, a2 * b0 - a0 * b2, a0 * b1 - a1 * b0], axis=1)


def _body(pos_ref, post_ref, nrm_ref, appr_ref, cidx_ref, *rest):
    wrefs = rest[:-2]
    grasp_ref, succ_ref = rest[-2:]

    p = pos_ref[0]
    pT = post_ref[0]

    sqc = jnp.sum(p * p, axis=1, keepdims=True)
    sqr = jnp.sum(pT * pT, axis=0, keepdims=True)
    d2 = sqc + sqr - 2.0 * _dot(p, pT)

    col = lax.broadcasted_iota(jnp.int32, (N, N), 1)
    idxs = []
    relks = []
    for _ in range(K):
        m = jnp.min(d2, axis=1, keepdims=True)
        ik = jnp.min(jnp.where(d2 == m, col, N), axis=1, keepdims=True)
        hot = col == ik
        d2 = jnp.where(hot, BIG, d2)
        idxs.append(ik)
        relks.append(_gdot(hot.astype(jnp.float32), p) - p)

    def take(n_layers, off):
        return [(wrefs[off + 2 * i], wrefs[off + 2 * i + 1])
                for i in range(n_layers)], off + 2 * n_layers

    off = 0
    conv1, off = take(2, off)
    conv2, off = take(2, off)
    conv3, off = take(2, off)
    gmlp1, off = take(3, off)
    gmlp2, off = take(2, off)
    clsw, off = take(4, off)

    def mlp(x, layers):
        n = len(layers)
        for i, (w, b) in enumerate(layers):
            x = _dot(x, w[:]) + b[:]
            if i < n - 1:
                x = jnp.maximum(x, 0.0)
        return x

    def pconv(x, fin, layers):
        (w1, b1), (w2, b2) = layers
        w1a = w1[:]
        w1x, w1r = w1a[:fin, :], w1a[fin:, :]
        t = _dot(x, w1x) + b1[:]
        w2a = w2[:]
        acc = None
        for k in range(K):
            a = (col == idxs[k]).astype(jnp.float32)
            y = jnp.maximum(_gdot(a, t) + _dot(relks[k], w1r), 0.0)
            msg = _dot(y, w2a)
            acc = msg if acc is None else jnp.maximum(acc, msg)
        return jnp.maximum(acc + b2[:], 0.0)

    h1 = pconv(p, 3, conv1)
    h2 = pconv(h1, 32, conv2)
    h3 = pconv(h2, 64, conv3)
    contact_emd = jnp.concatenate([h1, h2, h3], axis=1)

    g = mlp(contact_emd, gmlp1)
    gmax = jnp.max(g, axis=0, keepdims=True)
    g2 = mlp(jnp.concatenate(
        [contact_emd, jnp.broadcast_to(gmax, (N, gmax.shape[1]))], axis=1),
        gmlp2)
    global_emd = jnp.max(g2, axis=0, keepdims=True)

    cid = cidx_ref[0]
    ccol = lax.broadcasted_iota(jnp.int32, (NUM_CONTACT, N), 1)
    cmat = (ccol == cid).astype(jnp.float32)
    ce = _gdot(cmat, contact_emd)
    ef = jnp.concatenate(
        [ce, jnp.broadcast_to(global_emd, (NUM_CONTACT, global_emd.shape[1]))],
        axis=1)
    succ_ref[0] = mlp(ef, clsw)

    cpts = _gdot(cmat, p)
    cnrm = _gdot(cmat, nrm_ref[0])
    ap = appr_ref[0]
    rel = ap - cpts
    rel_n = _normalize(rel)
    cn = _normalize(cnrm)
    x_axis = _normalize(_cross(cn, rel_n))
    approach = -_normalize(_cross(x_axis, cn))
    dot = -jnp.sum(rel * approach, axis=1, keepdims=True)
    shift = GRIPPER_DEPTH + dot
    gpos = ap - shift * approach

    zeros = jnp.zeros((NUM_CONTACT, 1), jnp.float32)
    ones = jnp.ones((NUM_CONTACT, 1), jnp.float32)
    cols16 = []
    for i in range(3):
        cols16 += [cn[:, i:i + 1], x_axis[:, i:i + 1],
                   approach[:, i:i + 1], gpos[:, i:i + 1]]
    cols16 += [zeros, zeros, zeros, ones]
    grasp_ref[0] = jnp.concatenate(cols16, axis=1)


def kernel(pos, normals, appr_points, params, contact_idx):
    B = pos.shape[0]
    posT = jnp.swapaxes(pos, 1, 2)
    appr3 = appr_points.reshape(B, 1, 3)
    cidx3 = contact_idx.reshape(B, NUM_CONTACT, 1).astype(jnp.int32)

    wlist = []
    for name in ('conv1', 'conv2', 'conv3', 'gmlp1', 'gmlp2', 'cls'):
        for (w, b) in params[name]:
            wlist.append(w)
            wlist.append(b.reshape(1, -1))

    def ball_spec(shape):
        return pl.BlockSpec((1,) + shape, lambda b: (b, 0, 0))

    def const_spec(arr):
        return pl.BlockSpec(arr.shape, lambda b: (0, 0))

    in_specs = [
        ball_spec((N, 3)),
        ball_spec((3, N)),
        ball_spec((N, 3)),
        ball_spec((1, 3)),
        ball_spec((NUM_CONTACT, 1)),
    ] + [const_spec(w) for w in wlist]

    out_specs = [
        ball_spec((NUM_CONTACT, 16)),
        ball_spec((NUM_CONTACT, 1)),
    ]
    out_shapes = [
        jax.ShapeDtypeStruct((B, NUM_CONTACT, 16), jnp.float32),
        jax.ShapeDtypeStruct((B, NUM_CONTACT, 1), jnp.float32),
    ]

    grasp16, succ = pl.pallas_call(
        _body,
        grid=(B,),
        in_specs=in_specs,
        out_specs=out_specs,
        out_shape=out_shapes,
        compiler_params=pltpu.CompilerParams(
            dimension_semantics=("arbitrary",),
            vmem_limit_bytes=120 * 1024 * 1024),
    )(pos, posT, normals, appr3, cidx3, *wlist)

    return grasp16.reshape(B, NUM_CONTACT, 4, 4), succ.reshape(B, NUM_CONTACT)


# speedup vs baseline: 25.5856x; 1.6026x over previous
import functools

import jax
import jax.numpy as jnp
from jax import lax
from jax.experimental import pallas as pl
from jax.experimental.pallas import tpu as pltpu

N = 512
K = 16
NUM_CONTACT = 32
GRIPPER_DEPTH = 0.072 - 0.007
BIG = 1e30


def _dot(a, b):
    return jnp.dot(a, b, preferred_element_type=jnp.float32)


def _gdot(a, b):
    hi = b.astype(jnp.bfloat16).astype(jnp.float32)
    return _dot(a, hi) + _dot(a, b - hi)


def _normalize(v):
    n = jnp.sqrt(jnp.sum(v * v, axis=1, keepdims=True))
    return v / jnp.maximum(n, 1e-12)


def _cross(a, b):
    a0, a1, a2 = a[:, 0:1], a[:, 1:2], a[:, 2:3]
    b0, b1, b2 = b[:, 0:1], b[:, 1:2], b[:, 2:3]
    return jnp.concatenate(
        [a1 * b2 - a2 * b1, a2 * b0 - a0 * b2, a0 * b1 - a1 * b0], axis=1)


def _body(pos_ref, post_ref, nrm_ref, appr_ref, cidx_ref, *rest):
    wrefs = rest[:-2]
    grasp_ref, succ_ref = rest[-2:]

    p = pos_ref[0]
    pT = post_ref[0]

    sqc = jnp.sum(p * p, axis=1, keepdims=True)
    sqr = jnp.sum(pT * pT, axis=0, keepdims=True)
    d2 = sqc + sqr - 2.0 * _dot(p, pT)

    col = lax.broadcasted_iota(jnp.int32, (N, N), 1)
    idxs = []
    for _ in range(K):
        m = jnp.min(d2, axis=1, keepdims=True)
        ik = jnp.min(jnp.where(d2 == m, col, N), axis=1, keepdims=True)
        hot = col == ik
        d2 = jnp.where(hot, BIG, d2)
        idxs.append(ik)

    idxcat = jnp.concatenate(idxs, axis=0)
    ecol = lax.broadcasted_iota(jnp.int32, (K * N, N), 1)
    ab = (ecol == idxcat).astype(jnp.bfloat16)

    p_hi = p.astype(jnp.bfloat16)
    p_lo = (p - p_hi.astype(jnp.float32)).astype(jnp.bfloat16)
    gp = _dot(ab, jnp.concatenate([p_hi, p_lo], axis=1))
    p_src = gp[:, :3] + gp[:, 3:]
    p_tile = jnp.concatenate([p] * K, axis=0)
    rel = p_src - p_tile

    def take(n_layers, off):
        return [(wrefs[off + 2 * i], wrefs[off + 2 * i + 1])
                for i in range(n_layers)], off + 2 * n_layers

    off = 0
    conv1, off = take(2, off)
    conv2, off = take(2, off)
    conv3, off = take(2, off)
    gmlp1, off = take(3, off)
    gmlp2, off = take(2, off)
    clsw, off = take(4, off)

    def mlp(x, layers):
        n = len(layers)
        for i, (w, b) in enumerate(layers):
            x = _dot(x, w[:]) + b[:]
            if i < n - 1:
                x = jnp.maximum(x, 0.0)
        return x

    def pconv(xg, layers):
        (w1, b1), (w2, b2) = layers
        y = jnp.maximum(
            _dot(jnp.concatenate([xg, rel], axis=1), w1[:]) + b1[:], 0.0)
        msg = _dot(y, w2[:])
        seg = jnp.max(msg.reshape(K, N, msg.shape[1]), axis=0)
        return jnp.maximum(seg + b2[:], 0.0)

    h1 = pconv(p_src, conv1)
    h2 = pconv(_dot(ab, h1.astype(jnp.bfloat16)), conv2)
    h3 = pconv(_dot(ab, h2.astype(jnp.bfloat16)), conv3)
    contact_emd = jnp.concatenate([h1, h2, h3], axis=1)

    g = mlp(contact_emd, gmlp1)
    gmax = jnp.max(g, axis=0, keepdims=True)
    g2 = mlp(jnp.concatenate(
        [contact_emd, jnp.broadcast_to(gmax, (N, gmax.shape[1]))], axis=1),
        gmlp2)
    global_emd = jnp.max(g2, axis=0, keepdims=True)

    cid = cidx_ref[0]
    ccol = lax.broadcasted_iota(jnp.int32, (NUM_CONTACT, N), 1)
    cmat = (ccol == cid).astype(jnp.float32)
    ce = _gdot(cmat, contact_emd)
    ef = jnp.concatenate(
        [ce, jnp.broadcast_to(global_emd, (NUM_CONTACT, global_emd.shape[1]))],
        axis=1)
    succ_ref[0] = mlp(ef, clsw)

    cpts = _gdot(cmat, p)
    cnrm = _gdot(cmat, nrm_ref[0])
    ap = appr_ref[0]
    rel = ap - cpts
    rel_n = _normalize(rel)
    cn = _normalize(cnrm)
    x_axis = _normalize(_cross(cn, rel_n))
    approach = -_normalize(_cross(x_axis, cn))
    dot = -jnp.sum(rel * approach, axis=1, keepdims=True)
    shift = GRIPPER_DEPTH + dot
    gpos = ap - shift * approach

    zeros = jnp.zeros((NUM_CONTACT, 1), jnp.float32)
    ones = jnp.ones((NUM_CONTACT, 1), jnp.float32)
    cols16 = []
    for i in range(3):
        cols16 += [cn[:, i:i + 1], x_axis[:, i:i + 1],
                   approach[:, i:i + 1], gpos[:, i:i + 1]]
    cols16 += [zeros, zeros, zeros, ones]
    grasp_ref[0] = jnp.concatenate(cols16, axis=1)


def kernel(pos, normals, appr_points, params, contact_idx):
    B = pos.shape[0]
    posT = jnp.swapaxes(pos, 1, 2)
    appr3 = appr_points.reshape(B, 1, 3)
    cidx3 = contact_idx.reshape(B, NUM_CONTACT, 1).astype(jnp.int32)

    wlist = []
    for name in ('conv1', 'conv2', 'conv3', 'gmlp1', 'gmlp2', 'cls'):
        for (w, b) in params[name]:
            wlist.append(w)
            wlist.append(b.reshape(1, -1))

    def ball_spec(shape):
        return pl.BlockSpec((1,) + shape, lambda b: (b, 0, 0))

    def const_spec(arr):
        return pl.BlockSpec(arr.shape, lambda b: (0, 0))

    in_specs = [
        ball_spec((N, 3)),
        ball_spec((3, N)),
        ball_spec((N, 3)),
        ball_spec((1, 3)),
        ball_spec((NUM_CONTACT, 1)),
    ] + [const_spec(w) for w in wlist]

    out_specs = [
        ball_spec((NUM_CONTACT, 16)),
        ball_spec((NUM_CONTACT, 1)),
    ]
    out_shapes = [
        jax.ShapeDtypeStruct((B, NUM_CONTACT, 16), jnp.float32),
        jax.ShapeDtypeStruct((B, NUM_CONTACT, 1), jnp.float32),
    ]

    grasp16, succ = pl.pallas_call(
        _body,
        grid=(B,),
        in_specs=in_specs,
        out_specs=out_specs,
        out_shape=out_shapes,
        compiler_params=pltpu.CompilerParams(
            dimension_semantics=("arbitrary",),
            vmem_limit_bytes=120 * 1024 * 1024),
    )(pos, posT, normals, appr3, cidx3, *wlist)

    return grasp16.reshape(B, NUM_CONTACT, 4, 4), succ.reshape(B, NUM_CONTACT)
